# baseline (speedup 1.0000x reference)
import functools

import jax
import jax.numpy as jnp
from jax import lax
from jax.experimental import pallas as pl
from jax.experimental.pallas import tpu as pltpu


def _score_topk_body(k_ref, bias_ref, hid_ref, w_ref, scores_ref, mask_ref,
                     *, t_blk, n_t, t_total):
    b = pl.program_id(0)
    t = pl.program_id(1)

    part = lax.dot_general(
        w_ref[...], hid_ref[0],
        dimension_numbers=(((1,), (1,)), ((), ())),
        preferred_element_type=jnp.float32,
    )
    scores_ref[0, :, pl.ds(t * t_blk, t_blk)] = part + bias_ref[0]

    @pl.when(t == n_t - 1)
    def _select():
        s = scores_ref[0]
        u = lax.bitcast_convert_type(s, jnp.uint32)
        neg = u >= jnp.uint32(0x80000000)
        key = jnp.where(neg, ~u, u | jnp.uint32(0x80000000))
        kk = k_ref[b]

        def bit_step(i, th):
            cand = th | (jnp.uint32(1) << (31 - i).astype(jnp.uint32))
            cnt = jnp.sum((key >= cand).astype(jnp.int32))
            return jnp.where(cnt >= kk, cand, th)

        th = lax.fori_loop(0, 32, bit_step, jnp.uint32(0), unroll=True)

        gt = key > th
        tie = key == th
        need = kk - jnp.sum(gt.astype(jnp.int32))
        idxs = lax.broadcasted_iota(jnp.int32, s.shape, 1)

        def idx_step(i, r):
            cand = r + (jnp.int32(1) << (11 - i))
            cnt = jnp.sum((tie & (idxs < cand)).astype(jnp.int32))
            return jnp.where(cnt < need, cand, r)

        r = lax.fori_loop(0, 12, idx_step, jnp.int32(0), unroll=True)

        mask = gt | (tie & (idxs <= r))
        mask_ref[0] = mask.astype(jnp.int8)


@functools.partial(jax.jit, static_argnames=())
def kernel(hidden, keep_ratio, W, b):
    B, T, C = hidden.shape
    T_BLK = 512
    N_T = T // T_BLK

    kr = jnp.clip(keep_ratio, 0.1, 1.0)
    k = jnp.maximum(1, (kr * T).astype(jnp.int32))
    w_row = W.reshape(1, C)

    grid = (B, N_T)
    scores, mask_i8 = pl.pallas_call(
        functools.partial(_score_topk_body, t_blk=T_BLK, n_t=N_T, t_total=T),
        grid=grid,
        in_specs=[
            pl.BlockSpec(memory_space=pltpu.SMEM),
            pl.BlockSpec(memory_space=pltpu.SMEM),
            pl.BlockSpec((1, T_BLK, C), lambda b_, t_: (b_, t_, 0)),
            pl.BlockSpec((1, C), lambda b_, t_: (0, 0)),
        ],
        out_specs=[
            pl.BlockSpec((1, 1, T), lambda b_, t_: (b_, 0, 0)),
            pl.BlockSpec((1, 1, T), lambda b_, t_: (b_, 0, 0)),
        ],
        out_shape=[
            jax.ShapeDtypeStruct((B, 1, T), jnp.float32),
            jax.ShapeDtypeStruct((B, 1, T), jnp.int8),
        ],
        compiler_params=pltpu.CompilerParams(
            dimension_semantics=("arbitrary", "arbitrary"),
        ),
    )(k, b, hidden, w_row)

    return (mask_i8.reshape(B, T).astype(jnp.bool_), scores.reshape(B, T))


# speedup vs baseline: 1.1849x; 1.1849x over previous
import functools

import jax
import jax.numpy as jnp
from jax import lax
from jax.experimental import pallas as pl
from jax.experimental.pallas import tpu as pltpu


def _scan_body(bias_ref, hid_ref, w_ref, scores_ref):
    part = lax.dot_general(
        w_ref[...], hid_ref[...],
        dimension_numbers=(((1,), (1,)), ((), ())),
        preferred_element_type=jnp.float32,
    )
    scores_ref[0] = part + bias_ref[0]


def _select_body(k_ref, scores_ref, mask_ref):
    s = scores_ref[...]
    B = s.shape[0]
    u = lax.bitcast_convert_type(s, jnp.uint32)
    neg = u >= jnp.uint32(0x80000000)
    key = jnp.where(neg, ~u, u | jnp.uint32(0x80000000))
    kk = k_ref[...]

    def bit_step(i, th):
        cand = th | (jnp.uint32(1) << (31 - i).astype(jnp.uint32))
        cnt = jnp.sum((key >= cand).astype(jnp.int32), axis=1, keepdims=True)
        return jnp.where(cnt >= kk, cand, th)

    th = lax.fori_loop(0, 32, bit_step, jnp.zeros((B, 1), jnp.uint32),
                       unroll=True)

    gt = key > th
    tie = key == th
    need = kk - jnp.sum(gt.astype(jnp.int32), axis=1, keepdims=True)
    idxs = lax.broadcasted_iota(jnp.int32, s.shape, 1)

    def idx_step(i, r):
        cand = r + (jnp.int32(1) << (11 - i))
        cnt = jnp.sum((tie & (idxs < cand)).astype(jnp.int32),
                      axis=1, keepdims=True)
        return jnp.where(cnt < need, cand, r)

    r = lax.fori_loop(0, 12, idx_step, jnp.zeros((B, 1), jnp.int32),
                      unroll=True)

    mask_ref[...] = (gt | (tie & (idxs <= r))).astype(jnp.int8)


def kernel(hidden, keep_ratio, W, b):
    B, T, C = hidden.shape
    T_BLK = 512
    N = (B * T) // T_BLK

    kr = jnp.clip(keep_ratio, 0.1, 1.0)
    k = jnp.maximum(1, (kr * T).astype(jnp.int32))
    w_row = W.reshape(1, C)
    hid2d = hidden.reshape(B * T, C)

    scores3 = pl.pallas_call(
        _scan_body,
        grid=(N,),
        in_specs=[
            pl.BlockSpec(memory_space=pltpu.SMEM),
            pl.BlockSpec((T_BLK, C), lambda i: (i, 0)),
            pl.BlockSpec((1, C), lambda i: (0, 0)),
        ],
        out_specs=pl.BlockSpec((1, 1, T_BLK), lambda i: (i, 0, 0)),
        out_shape=jax.ShapeDtypeStruct((N, 1, T_BLK), jnp.float32),
        compiler_params=pltpu.CompilerParams(
            dimension_semantics=("arbitrary",),
        ),
    )(b, hid2d, w_row)
    scores = scores3.reshape(B, T)

    mask_i8 = pl.pallas_call(
        _select_body,
        in_specs=[
            pl.BlockSpec((B, 1), lambda: (0, 0)),
            pl.BlockSpec((B, T), lambda: (0, 0)),
        ],
        out_specs=pl.BlockSpec((B, T), lambda: (0, 0)),
        out_shape=jax.ShapeDtypeStruct((B, T), jnp.int8),
    )(k.reshape(B, 1), scores)

    return (mask_i8.astype(jnp.bool_), scores)
